# baseline (speedup 1.0000x reference)
import functools

import jax
import jax.numpy as jnp
from jax.experimental import pallas as pl
from jax.experimental.pallas import tpu as pltpu

_R2 = 0.1 * 0.1
_NS = 16.0
_CH = 32


def _end_loss_kernel(p1a_ref, bsq_ref, p2last_ref, p4_ref, e_ref,
                     et_ref, out_ref):
    b = pl.program_id(0)
    s = pl.program_id(1)

    p1a = p1a_ref[0]
    bsq = bsq_ref[0]
    last = p2last_ref[0]
    p4 = p4_ref[0]
    e = e_ref[...]
    et = et_ref[...]

    n = p4.shape[0]
    nc = e.shape[1]

    p1m2 = p1a[:, 0:3]
    asq = p1a[:, 3:4]
    d = jax.lax.dot_general(
        p1m2, p4[:, 0:3], (((1,), (1,)), ((), ())),
        preferred_element_type=jnp.float32)
    d = d + asq
    d = d + bsq

    m = jnp.where(d <= _R2, 1.0, 0.0).astype(jnp.bfloat16)

    lane = jax.lax.broadcasted_iota(jnp.int32, (1, n), 1)
    sub = lane & (_CH - 1)
    c = m
    k = 1
    while k < _CH:
        mask = jnp.where(sub >= k, 1.0, 0.0).astype(jnp.bfloat16)
        c = c + pltpu.roll(c, k, 1) * mask
        k *= 2

    tot = jax.lax.dot_general(
        m, e, (((1,), (0,)), ((), ())),
        preferred_element_type=jnp.float32)
    lane_c = jax.lax.broadcasted_iota(jnp.int32, (1, nc), 1)
    p = tot
    k = 1
    while k < nc:
        maskc = jnp.where(lane_c >= k, 1.0, 0.0)
        p = p + pltpu.roll(p, k, 1) * maskc
        k *= 2
    p_excl = p - tot
    cfull = c + jax.lax.dot_general(
        p_excl, et, (((1,), (0,)), ((), ())),
        preferred_element_type=jnp.float32).astype(jnp.bfloat16)

    zero = jnp.bfloat16(0.0)
    w = jnp.where(cfull <= jnp.bfloat16(_NS), m, zero)
    f = jnp.where(cfull == jnp.bfloat16(1.0), m, zero)

    wf = jnp.concatenate([w, f], axis=0)
    sums = jax.lax.dot_general(
        wf, p4, (((1,), (0,)), ((), ())),
        preferred_element_type=jnp.float32)

    sb = p1a.shape[0]
    sel = sums[:sb, 0:3]
    cnt = sums[:sb, 3:4]
    fst = sums[sb:, 0:3]
    has = sums[sb:, 3:4]

    sum_vec = sel + (_NS - cnt) * fst + (1.0 - has) * _NS * last + 8.0 * p1m2
    dist = jnp.sqrt(jnp.sum(sum_vec * sum_vec, axis=1, keepdims=True))

    partial = jnp.sum(dist, axis=(0, 1), keepdims=True)

    @pl.when((b == 0) & (s == 0))
    def _():
        out_ref[...] = jnp.zeros_like(out_ref)

    out_ref[...] += partial


@jax.jit
def kernel(recon_points, gt_points):
    B, S, C = recon_points.shape
    N = gt_points.shape[1]
    SB = 512
    NC = N // _CH

    p1a = jnp.concatenate(
        [-2.0 * recon_points,
         jnp.sum(recon_points * recon_points, axis=2, keepdims=True)],
        axis=2)
    bsq = jnp.sum(gt_points * gt_points, axis=2)[:, None, :]
    gt_last = gt_points[:, N - 1:N, :]
    p4 = jnp.concatenate(
        [gt_points, jnp.ones((B, N, 1), jnp.float32)], axis=2)
    chunk_id = jnp.arange(N, dtype=jnp.int32) // _CH
    ef = (chunk_id[:, None] == jnp.arange(NC, dtype=jnp.int32)[None, :]
          ).astype(jnp.float32)
    e = ef.astype(jnp.bfloat16)
    et = ef.T

    total = pl.pallas_call(
        _end_loss_kernel,
        grid=(B, S // SB),
        in_specs=[
            pl.BlockSpec((1, SB, 4), lambda b, s: (b, s, 0)),
            pl.BlockSpec((1, 1, N), lambda b, s: (b, 0, 0)),
            pl.BlockSpec((1, 1, C), lambda b, s: (b, 0, 0)),
            pl.BlockSpec((1, N, 4), lambda b, s: (b, 0, 0)),
            pl.BlockSpec((N, NC), lambda b, s: (0, 0)),
            pl.BlockSpec((NC, N), lambda b, s: (0, 0)),
        ],
        out_specs=pl.BlockSpec((1, 1), lambda b, s: (0, 0)),
        out_shape=jax.ShapeDtypeStruct((1, 1), jnp.float32),
    )(p1a, bsq, gt_last, p4, e, et)

    mean_dist = total[0, 0] / (B * S)
    return mean_dist / S * 24


# speedup vs baseline: 1.0262x; 1.0262x over previous
import functools

import jax
import jax.numpy as jnp
from jax.experimental import pallas as pl
from jax.experimental.pallas import tpu as pltpu

_R2 = 0.1 * 0.1
_NS = 16.0
_CH = 16


def _end_loss_kernel(p1a_ref, p2a_ref, bsq_ref, p2last_ref, p4_ref, e_ref,
                     et_ref, out_ref):
    b = pl.program_id(0)
    s = pl.program_id(1)

    p1a = p1a_ref[0]
    p2a = p2a_ref[0]
    bsq = bsq_ref[0]
    last = p2last_ref[0]
    p4 = p4_ref[0]
    e = e_ref[...]
    et = et_ref[...]

    n = p2a.shape[1]
    nc = e.shape[1]

    p1 = p1a[:, 0:3]
    asq = p1a[:, 3:4]
    d = jax.lax.dot_general(
        p1, p2a, (((1,), (0,)), ((), ())),
        preferred_element_type=jnp.float32)
    d = d + asq
    d = d + bsq

    m = jnp.where(d <= _R2, 1.0, 0.0).astype(jnp.bfloat16)

    lane = jax.lax.broadcasted_iota(jnp.int32, (1, n), 1)
    sub = lane & (_CH - 1)
    c = m
    k = 1
    while k < _CH:
        mask = jnp.where(sub >= k, 1.0, 0.0).astype(jnp.bfloat16)
        c = c + pltpu.roll(c, k, 1) * mask
        k *= 2

    tot = jax.lax.dot_general(
        m, e, (((1,), (0,)), ((), ())),
        preferred_element_type=jnp.float32)
    lane_c = jax.lax.broadcasted_iota(jnp.int32, (1, nc), 1)
    p = tot
    k = 1
    while k < nc:
        maskc = jnp.where(lane_c >= k, 1.0, 0.0)
        p = p + pltpu.roll(p, k, 1) * maskc
        k *= 2
    p_excl = p - tot
    cfull = c + jax.lax.dot_general(
        p_excl, et, (((1,), (0,)), ((), ())),
        preferred_element_type=jnp.float32).astype(jnp.bfloat16)

    zero = jnp.bfloat16(0.0)
    w = jnp.where(cfull <= jnp.bfloat16(_NS), m, zero)
    f = jnp.where(cfull == jnp.bfloat16(1.0), m, zero)

    wf = jnp.concatenate([w, f], axis=0)
    sums = jax.lax.dot_general(
        wf, p4, (((1,), (0,)), ((), ())),
        preferred_element_type=jnp.float32)

    sb = p1a.shape[0]
    sel = sums[:sb, 0:3]
    cnt = sums[:sb, 3:4]
    fst = sums[sb:, 0:3]
    has = sums[sb:, 3:4]

    sum_vec = sel + (_NS - cnt) * fst + (1.0 - has) * _NS * last - _NS * p1
    dist = jnp.sqrt(jnp.sum(sum_vec * sum_vec, axis=1, keepdims=True))

    partial = jnp.sum(dist, axis=(0, 1), keepdims=True)

    @pl.when((b == 0) & (s == 0))
    def _():
        out_ref[...] = jnp.zeros_like(out_ref)

    out_ref[...] += partial


@jax.jit
def kernel(recon_points, gt_points):
    B, S, C = recon_points.shape
    N = gt_points.shape[1]
    SB = 512
    NC = N // _CH

    p1a = jnp.concatenate(
        [recon_points,
         jnp.sum(recon_points * recon_points, axis=2, keepdims=True)],
        axis=2)
    p2a = -2.0 * gt_points.transpose(0, 2, 1)
    bsq = jnp.sum(gt_points * gt_points, axis=2)[:, None, :]
    gt_last = gt_points[:, N - 1:N, :]
    p4 = jnp.concatenate(
        [gt_points, jnp.ones((B, N, 1), jnp.float32)], axis=2)
    chunk_id = jnp.arange(N, dtype=jnp.int32) // _CH
    ef = (chunk_id[:, None] == jnp.arange(NC, dtype=jnp.int32)[None, :]
          ).astype(jnp.float32)
    e = ef.astype(jnp.bfloat16)
    et = ef.T

    total = pl.pallas_call(
        _end_loss_kernel,
        grid=(B, S // SB),
        in_specs=[
            pl.BlockSpec((1, SB, 4), lambda b, s: (b, s, 0)),
            pl.BlockSpec((1, 3, N), lambda b, s: (b, 0, 0)),
            pl.BlockSpec((1, 1, N), lambda b, s: (b, 0, 0)),
            pl.BlockSpec((1, 1, C), lambda b, s: (b, 0, 0)),
            pl.BlockSpec((1, N, 4), lambda b, s: (b, 0, 0)),
            pl.BlockSpec((N, NC), lambda b, s: (0, 0)),
            pl.BlockSpec((NC, N), lambda b, s: (0, 0)),
        ],
        out_specs=pl.BlockSpec((1, 1), lambda b, s: (0, 0)),
        out_shape=jax.ShapeDtypeStruct((1, 1), jnp.float32),
    )(p1a, p2a, bsq, gt_last, p4, e, et)

    mean_dist = total[0, 0] / (B * S)
    return mean_dist / S * 24


# speedup vs baseline: 1.0331x; 1.0067x over previous
import jax
import jax.numpy as jnp
from jax.experimental import pallas as pl
from jax.experimental.pallas import tpu as pltpu

_R2 = 0.1 * 0.1
_NS = 16.0
_CH = 16


def _end_loss_kernel(p1a_ref, p2a_ref, bsq_ref, p2last_ref, p4_ref, e_ref,
                     et_ref, out_ref):
    b = pl.program_id(0)
    s = pl.program_id(1)

    p1a = p1a_ref[0]
    p2a = p2a_ref[0]
    bsq = bsq_ref[0]
    last = p2last_ref[0]
    p4 = p4_ref[0]
    e = e_ref[...]
    et = et_ref[...]

    n = p2a.shape[1]
    nc = e.shape[1]

    p1 = p1a[:, 0:3]
    asq = p1a[:, 3:4]
    d = jax.lax.dot_general(
        p1, p2a, (((1,), (0,)), ((), ())),
        preferred_element_type=jnp.float32)
    d = d + asq
    d = d + bsq

    m = jnp.where(d <= _R2, 1.0, 0.0).astype(jnp.bfloat16)

    lane = jax.lax.broadcasted_iota(jnp.int32, (1, n), 1)
    sub = lane & (_CH - 1)
    c = m
    k = 1
    while k < _CH:
        mask = jnp.where(sub >= k, 1.0, 0.0).astype(jnp.bfloat16)
        c = c + pltpu.roll(c, k, 1) * mask
        k *= 2

    tot = jax.lax.dot_general(
        m, e, (((1,), (0,)), ((), ())),
        preferred_element_type=jnp.float32)
    lane_c = jax.lax.broadcasted_iota(jnp.int32, (1, nc), 1)
    p = tot
    k = 1
    while k < nc:
        maskc = jnp.where(lane_c >= k, 1.0, 0.0)
        p = p + pltpu.roll(p, k, 1) * maskc
        k *= 2
    p_excl = p - tot
    cfull = c + jax.lax.dot_general(
        p_excl, et, (((1,), (0,)), ((), ())),
        preferred_element_type=jnp.float32).astype(jnp.bfloat16)

    zero = jnp.bfloat16(0.0)
    w = jnp.where(cfull <= jnp.bfloat16(_NS), m, zero)
    f = jnp.where(cfull == jnp.bfloat16(1.0), m, zero)

    wf = jnp.concatenate([w, f], axis=0)
    sums = jax.lax.dot_general(
        wf, p4, (((1,), (0,)), ((), ())),
        preferred_element_type=jnp.float32)

    sb = p1a.shape[0]
    sel = sums[:sb, 0:3]
    cnt = sums[:sb, 3:4]
    fst = sums[sb:, 0:3]
    has = sums[sb:, 3:4]

    sum_vec = sel + (_NS - cnt) * fst + (1.0 - has) * _NS * last - _NS * p1
    dist = jnp.sqrt(jnp.sum(sum_vec * sum_vec, axis=1, keepdims=True))

    partial = jnp.sum(dist, axis=(0, 1), keepdims=True)

    @pl.when((b == 0) & (s == 0))
    def _():
        out_ref[...] = jnp.zeros_like(out_ref)

    out_ref[...] += partial


@jax.jit
def kernel(recon_points, gt_points):
    B, S, C = recon_points.shape
    N = gt_points.shape[1]
    SB = 512
    NC = N // _CH

    p1a = jnp.concatenate(
        [recon_points,
         jnp.sum(recon_points * recon_points, axis=2, keepdims=True)],
        axis=2)
    p2a = -2.0 * gt_points.transpose(0, 2, 1)
    bsq = jnp.sum(gt_points * gt_points, axis=2)[:, None, :]
    gt_last = gt_points[:, N - 1:N, :]
    p4 = jnp.concatenate(
        [gt_points, jnp.ones((B, N, 1), jnp.float32)], axis=2)
    chunk_id = jnp.arange(N, dtype=jnp.int32) // _CH
    ef = (chunk_id[:, None] == jnp.arange(NC, dtype=jnp.int32)[None, :]
          ).astype(jnp.float32)
    e = ef.astype(jnp.bfloat16)
    et = ef.T

    total = pl.pallas_call(
        _end_loss_kernel,
        grid=(B, S // SB),
        in_specs=[
            pl.BlockSpec((1, SB, 4), lambda b, s: (b, s, 0)),
            pl.BlockSpec((1, 3, N), lambda b, s: (b, 0, 0)),
            pl.BlockSpec((1, 1, N), lambda b, s: (b, 0, 0)),
            pl.BlockSpec((1, 1, C), lambda b, s: (b, 0, 0)),
            pl.BlockSpec((1, N, 4), lambda b, s: (b, 0, 0)),
            pl.BlockSpec((N, NC), lambda b, s: (0, 0)),
            pl.BlockSpec((NC, N), lambda b, s: (0, 0)),
        ],
        out_specs=pl.BlockSpec((1, 1), lambda b, s: (0, 0)),
        out_shape=jax.ShapeDtypeStruct((1, 1), jnp.float32),
    )(p1a, p2a, bsq, gt_last, p4, e, et)

    mean_dist = total[0, 0] / (B * S)
    return mean_dist / S * 24
